# baseline (speedup 1.0000x reference)
import functools

import jax
import jax.numpy as jnp
from jax.experimental import pallas as pl

N = 10000
E = 160000
F = 1000
EF = 2000
D = 300
DE = 16
L = 5


def _mm_body(x_ref, w_ref, b_ref, o_ref):
    o_ref[...] = (
        jnp.dot(x_ref[...], w_ref[...], preferred_element_type=jnp.float32)
        + b_ref[...]
    )


def _mm(x, w, b, block_rows=1000):
    m, k = x.shape
    k2, n = w.shape
    assert k == k2
    grid = (m // block_rows,) if m % block_rows == 0 else None
    if grid is None:
        block_rows = m
        grid = (1,)
    return pl.pallas_call(
        _mm_body,
        grid=grid,
        in_specs=[
            pl.BlockSpec((block_rows, k), lambda i: (i, 0)),
            pl.BlockSpec((k, n), lambda i: (0, 0)),
            pl.BlockSpec((n,), lambda i: (0,)),
        ],
        out_specs=pl.BlockSpec((block_rows, n), lambda i: (i, 0)),
        out_shape=jax.ShapeDtypeStruct((m, n), jnp.float32),
    )(x, w, b)


def _gcn_conv(x, src, dst, e_emb, W, b, n_nodes, add_self_loop):
    h = _mm(x, W, b)
    deg = jnp.zeros((n_nodes,), h.dtype).at[dst].add(1.0)
    if add_self_loop:
        deg = deg + 1.0
    deg = jnp.maximum(deg, 1.0)
    dinv = jax.lax.rsqrt(deg)
    norm = dinv[src] * dinv[dst]
    msg = norm[:, None] * jax.nn.relu(h[src] + e_emb)
    out = jnp.zeros_like(h).at[dst].add(msg)
    if add_self_loop:
        out = out + h * (1.0 / deg)[:, None]
    return out


def _l2norm(v):
    n = jnp.sqrt(jnp.sum(v * v, axis=1, keepdims=True))
    return v / jnp.maximum(n, 1e-12)


def kernel(x, edge_index, edge_attr, frag_batch, frag_edge_index, dangling_edge_attr, enc_W, enc_b, enc_We, enc_be, proj_W1, proj_b1, proj_W2, proj_b2, pred_W, pred_b, pred_We, pred_be, cls_W, cls_b):
    src, dst = edge_index[0], edge_index[1]
    h = x
    for l in range(L):
        e_emb = _mm(edge_attr, enc_We[l], enc_be[l], block_rows=4000)
        h = _gcn_conv(h, src, dst, e_emb, enc_W[l], enc_b[l], N, True)
        if l < L - 1:
            h = jax.nn.relu(h)
    sums = jnp.zeros((F, D), h.dtype).at[frag_batch].add(h)
    cnt = jnp.zeros((F,), h.dtype).at[frag_batch].add(1.0)
    pooled = sums / jnp.maximum(cnt, 1.0)[:, None]
    out = _mm(jax.nn.relu(_mm(pooled, proj_W1, proj_b1)), proj_W2, proj_b2)
    f0 = _l2norm(out)
    u = jnp.concatenate([frag_edge_index[0], frag_edge_index[1]], axis=0)
    v = jnp.concatenate([frag_edge_index[1], frag_edge_index[0]], axis=0)
    uv_attr = jnp.concatenate([dangling_edge_attr, dangling_edge_attr], axis=0)
    e_emb = uv_attr @ pred_We + pred_be
    pred = _gcn_conv(out, u, v, e_emb, pred_W, pred_b, F, False)
    f1 = _l2norm(pred)
    f2 = jnp.roll(f1, 1, axis=0)
    pos = jnp.maximum(f0, f1) @ cls_W + cls_b
    neg = jnp.maximum(f0, f2) @ cls_W + cls_b
    logits = jnp.concatenate([pos, neg], axis=0).squeeze(1)
    labels = jnp.concatenate([jnp.ones((F,), jnp.float32), jnp.zeros((F,), jnp.float32)], axis=0)
    return (logits, labels)


# speedup vs baseline: 1.1542x; 1.1542x over previous
import functools

import jax
import jax.numpy as jnp
from jax import lax
from jax.experimental import pallas as pl
from jax.experimental.pallas import tpu as pltpu
from jax.experimental.pallas import tpu_sc as plsc

N = 10000
E = 160000
F = 1000
EF = 2000
D = 300
DE = 16
L = 5

NP = 10240
EP = 163840
FP = 1024
E2P = 4096
DP = 320
HQ = 80
NT = 16
RPT = NP // NT


def _tc_layer0(nrows):
    nb = nrows // 512

    def body(x_ref, w_ref, b_ref, ss_ref, gt_ref, it_ref):
        h = jnp.dot(x_ref[...], w_ref[0], preferred_element_type=jnp.float32)
        h = h + b_ref[0]
        gt_ref[0] = h
        it_ref[0] = h * ss_ref[...]

    return pl.pallas_call(
        body,
        grid=(4, nb),
        in_specs=[
            pl.BlockSpec((512, DP), lambda q, i: (i, 0)),
            pl.BlockSpec((1, DP, HQ), lambda q, i: (q, 0, 0)),
            pl.BlockSpec((1, 1, HQ), lambda q, i: (q, 0, 0)),
            pl.BlockSpec((512, 1), lambda q, i: (i, 0)),
        ],
        out_specs=[
            pl.BlockSpec((1, 512, HQ), lambda q, i: (q, i, 0)),
            pl.BlockSpec((1, 512, HQ), lambda q, i: (q, i, 0)),
        ],
        out_shape=[
            jax.ShapeDtypeStruct((4, nrows, HQ), jnp.float32),
            jax.ShapeDtypeStruct((4, nrows, HQ), jnp.float32),
        ],
    )


def _tc_layer(nrows):
    nb = nrows // 512

    def body(x0, x1, x2, x3, w0, w1, w2, w3, bias_ref, ss_ref, gt_ref, it_ref):
        xs = (x0, x1, x2, x3)
        ws = (w0, w1, w2, w3)
        h = bias_ref[0]
        for r in range(4):
            h = h + jnp.dot(
                jnp.maximum(xs[r][...], 0.0), ws[r][0, 0],
                preferred_element_type=jnp.float32,
            )
        gt_ref[0] = h
        it_ref[0] = h * ss_ref[...]

    def _xspec(r):
        return pl.BlockSpec((512, HQ), lambda q, i, r=r: (r * nb + i, 0))

    def _wspec(r):
        return pl.BlockSpec((1, 1, HQ, HQ), lambda q, i, r=r: (r, q, 0, 0))

    return pl.pallas_call(
        body,
        grid=(4, nb),
        in_specs=[
            _xspec(0), _xspec(1), _xspec(2), _xspec(3),
            _wspec(0), _wspec(1), _wspec(2), _wspec(3),
            pl.BlockSpec((1, 1, HQ), lambda q, i: (q, 0, 0)),
            pl.BlockSpec((512, 1), lambda q, i: (i, 0)),
        ],
        out_specs=[
            pl.BlockSpec((1, 512, HQ), lambda q, i: (q, i, 0)),
            pl.BlockSpec((1, 512, HQ), lambda q, i: (q, i, 0)),
        ],
        out_shape=[
            jax.ShapeDtypeStruct((4, nrows, HQ), jnp.float32),
            jax.ShapeDtypeStruct((4, nrows, HQ), jnp.float32),
        ],
    )


def _tc_eemb(nedges):
    nb = nedges // 512

    def body(ea_ref, we_ref, be_ref, o_ref):
        o_ref[0] = (
            jnp.dot(ea_ref[...], we_ref[0], preferred_element_type=jnp.float32)
            + be_ref[0]
        )

    return pl.pallas_call(
        body,
        grid=(4, nb),
        in_specs=[
            pl.BlockSpec((512, DE), lambda q, i: (i, 0)),
            pl.BlockSpec((1, DE, HQ), lambda q, i: (q, 0, 0)),
            pl.BlockSpec((1, 1, HQ), lambda q, i: (q, 0, 0)),
        ],
        out_specs=pl.BlockSpec((1, 512, HQ), lambda q, i: (q, i, 0)),
        out_shape=jax.ShapeDtypeStruct((4, nedges, HQ), jnp.float32),
    )


def _tc_prep():
    def body(degt_ref, fct_ref, fvt_ref, dinv_ref, ss_ref, rcnt_ref, dinv2_ref):
        d = degt_ref[0:NP, 0:1] + degt_ref[NP : 2 * NP, 0:1] + 1.0
        row = lax.broadcasted_iota(jnp.int32, (NP, 16), 0)
        dinv_ref[...] = jnp.where(row < N, lax.rsqrt(d), 0.0)
        ss_ref[...] = 1.0 / d
        cnt = fct_ref[0:FP, 0:1] + fct_ref[FP : 2 * FP, 0:1]
        rcnt_ref[...] = 1.0 / jnp.maximum(cnt, 1.0)
        dv = fvt_ref[0:FP, 0:1] + fvt_ref[FP : 2 * FP, 0:1]
        dinv2_ref[...] = jnp.broadcast_to(
            lax.rsqrt(jnp.maximum(dv, 1.0)), (FP, 16)
        )

    return pl.pallas_call(
        body,
        out_shape=[
            jax.ShapeDtypeStruct((NP, 16), jnp.float32),
            jax.ShapeDtypeStruct((NP, 1), jnp.float32),
            jax.ShapeDtypeStruct((FP, 1), jnp.float32),
            jax.ShapeDtypeStruct((FP, 16), jnp.float32),
        ],
    )


def _tc_mlp():
    def body(sums_ref, rcnt_ref, w1_ref, b1_ref, w2_ref, b2_ref, o_ref, f0_ref):
        rc = rcnt_ref[...]
        w1 = w1_ref[...]
        t = b1_ref[...]
        for q in range(4):
            t = t + jnp.dot(
                sums_ref[q] * rc, w1[q * HQ : (q + 1) * HQ],
                preferred_element_type=jnp.float32,
            )
        t = jnp.maximum(t, 0.0)
        o = jnp.dot(t, w2_ref[...], preferred_element_type=jnp.float32) + b2_ref[...]
        o_ref[...] = o
        nrm = jnp.sqrt(jnp.sum(o * o, axis=1, keepdims=True))
        f0_ref[...] = o / jnp.maximum(nrm, 1e-12)

    return pl.pallas_call(
        body,
        out_shape=[
            jax.ShapeDtypeStruct((FP, DP), jnp.float32),
            jax.ShapeDtypeStruct((FP, DP), jnp.float32),
        ],
    )


def _tc_final():
    def body(pred_ref, f0_ref, cw_ref, cb_ref, o_ref):
        p = jnp.concatenate(
            [pred_ref[0], pred_ref[1], pred_ref[2], pred_ref[3]], axis=1
        )
        nrm = jnp.sqrt(jnp.sum(p * p, axis=1, keepdims=True))
        f1 = p / jnp.maximum(nrm, 1e-12)
        f0 = f0_ref[...]
        f2 = jnp.concatenate(
            [f1[F - 1 : F], f1[0 : F - 1], jnp.zeros((FP - F, DP), jnp.float32)],
            axis=0,
        )
        cw = cw_ref[...]
        cb = cb_ref[...]
        pos = jnp.dot(jnp.maximum(f0, f1), cw, preferred_element_type=jnp.float32) + cb
        neg = jnp.dot(jnp.maximum(f0, f2), cw, preferred_element_type=jnp.float32) + cb
        o_ref[...] = jnp.concatenate([pos, neg], axis=1)

    return pl.pallas_call(
        body,
        out_shape=jax.ShapeDtypeStruct((FP, 2), jnp.float32),
    )


def _sc_mesh():
    return plsc.VectorSubcoreMesh(core_axis_name="c", subcore_axis_name="s")


_SC_PARAMS = pltpu.CompilerParams(use_tc_tiling_on_sc=False)


def _sc_hist(bins, nch, chk):
    rb = bins // NT

    @functools.partial(
        pl.kernel,
        out_type=jax.ShapeDtypeStruct((2 * bins, 16), jnp.float32),
        mesh=_sc_mesh(),
        compiler_params=_SC_PARAMS,
        scratch_types=[
            pltpu.VMEM((nch, chk), jnp.int32),
            pltpu.VMEM((chk, 16), jnp.float32),
            pltpu.VMEM_SHARED((bins, 16), jnp.float32),
        ],
    )
    def k(idx_hbm, tmpl_hbm, zeros_hbm, out_hbm, idxb, tmplb, tbl):
        c = lax.axis_index("c")
        s = lax.axis_index("s")
        w = c * NT + s

        @pl.when(s == 0)
        def _():
            pltpu.sync_copy(zeros_hbm, tbl)

        plsc.subcore_barrier()
        pltpu.sync_copy(idx_hbm.at[w], idxb)
        pltpu.sync_copy(tmpl_hbm.at[pl.ds(0, chk)], tmplb)

        def body(g, _):
            pltpu.sync_copy(tmplb, tbl.at[idxb.at[g]], add=True)
            return 0

        lax.fori_loop(0, nch, body, 0)
        plsc.subcore_barrier()
        pltpu.sync_copy(
            tbl.at[pl.ds(s * rb, rb)], out_hbm.at[pl.ds(c * bins + s * rb, rb)]
        )

    return k


def _sc_norm(nedges, nch, chk):
    eptw = nch * chk

    @functools.partial(
        pl.kernel,
        out_type=jax.ShapeDtypeStruct((nedges, 16), jnp.float32),
        mesh=_sc_mesh(),
        compiler_params=_SC_PARAMS,
        scratch_types=[
            pltpu.VMEM((nch, chk), jnp.int32),
            pltpu.VMEM((nch, chk), jnp.int32),
            pltpu.VMEM((chk, 16), jnp.float32),
            pltpu.VMEM((chk, 16), jnp.float32),
            pltpu.VMEM((chk, 16), jnp.float32),
            pltpu.SemaphoreType.DMA,
            pltpu.SemaphoreType.DMA,
        ],
    )
    def k(src_hbm, dst_hbm, dinv_hbm, out_hbm, srcb, dstb, srows, drows, nb,
          s1, s2):
        c = lax.axis_index("c")
        s = lax.axis_index("s")
        w = c * NT + s
        pltpu.sync_copy(src_hbm.at[w], srcb)
        pltpu.sync_copy(dst_hbm.at[w], dstb)

        def body(g, _):
            cp1 = pltpu.async_copy(dinv_hbm.at[srcb.at[g]], srows, s1)
            cp2 = pltpu.async_copy(dinv_hbm.at[dstb.at[g]], drows, s2)
            cp1.wait()
            cp2.wait()

            def row(e, _):
                nb[e] = srows[e] * drows[e]
                return 0

            lax.fori_loop(0, chk, row, 0)
            pltpu.sync_copy(nb, out_hbm.at[pl.ds(w * eptw + g * chk, chk)])
            return 0

        lax.fori_loop(0, nch, body, 0)

    return k


def _sc_edge(nrows, nedges, nch, chk):
    rb = nrows // NT
    ept = nch * chk

    @functools.partial(
        pl.kernel,
        out_type=jax.ShapeDtypeStruct((4 * nrows, HQ), jnp.float32),
        mesh=_sc_mesh(),
        compiler_params=_SC_PARAMS,
        scratch_types=[
            pltpu.VMEM((nch, chk), jnp.int32),
            pltpu.VMEM((nch, chk), jnp.int32),
            pltpu.VMEM((nch, chk), jnp.int32),
            pltpu.VMEM((chk, 16), jnp.float32),
            pltpu.VMEM((chk, HQ), jnp.float32),
            pltpu.VMEM((chk, HQ), jnp.float32),
            pltpu.VMEM((chk, HQ), jnp.float32),
            pltpu.VMEM_SHARED((nrows, HQ), jnp.float32),
            pltpu.SemaphoreType.DMA,
            pltpu.SemaphoreType.DMA,
            pltpu.SemaphoreType.DMA,
        ],
    )
    def k(gt_hbm, em_hbm, src_hbm, dst_hbm, nrm_hbm, init_hbm, out_hbm,
          srcb, gix, dix, nrmb, grow, erow, mout, tbl, s1, s2, s3):
        c = lax.axis_index("c")
        s = lax.axis_index("s")
        pltpu.sync_copy(src_hbm.at[s], srcb)
        pltpu.sync_copy(dst_hbm.at[s], dix)

        for q in range(2):
            qq = 2 * c + q
            pltpu.sync_copy(
                init_hbm.at[pl.ds(qq * nrows + s * rb, rb)],
                tbl.at[pl.ds(s * rb, rb)],
            )
            ov = jnp.zeros((16,), jnp.int32) + qq * nrows

            def adj(g, _):
                for kk in range(chk // 16):
                    sl = pl.ds(kk * 16, 16)
                    gix[g, sl] = srcb[g, sl] + ov
                return 0

            lax.fori_loop(0, nch, adj, 0)
            plsc.subcore_barrier()

            def chunk(g, _):
                cp1 = pltpu.async_copy(gt_hbm.at[gix.at[g]], grow, s1)
                cp2 = pltpu.async_copy(
                    em_hbm.at[pl.ds(qq * nedges + s * ept + g * chk, chk)],
                    erow, s2,
                )
                cp3 = pltpu.async_copy(
                    nrm_hbm.at[pl.ds(s * ept + g * chk, chk)], nrmb, s3
                )
                cp1.wait()
                cp2.wait()
                cp3.wait()

                def edge(e, _):
                    nv = nrmb[e]
                    for j in range(HQ // 16):
                        sl = pl.ds(j * 16, 16)
                        mout[e, sl] = nv * jnp.maximum(
                            grow[e, sl] + erow[e, sl], 0.0
                        )
                    return 0

                lax.fori_loop(0, chk, edge, 0)
                pltpu.sync_copy(mout, tbl.at[dix.at[g]], add=True)
                return 0

            lax.fori_loop(0, nch, chunk, 0)
            plsc.subcore_barrier()
            pltpu.sync_copy(
                tbl.at[pl.ds(s * rb, rb)],
                out_hbm.at[pl.ds(qq * nrows + s * rb, rb)],
            )

    return k


def _sc_pool():
    rbf = FP // NT

    @functools.partial(
        pl.kernel,
        out_type=jax.ShapeDtypeStruct((4 * FP, HQ), jnp.float32),
        mesh=_sc_mesh(),
        compiler_params=_SC_PARAMS,
        scratch_types=[
            pltpu.VMEM((5, 128), jnp.int32),
            pltpu.VMEM((128, HQ), jnp.float32),
            pltpu.VMEM_SHARED((FP, HQ), jnp.float32),
            pltpu.SemaphoreType.DMA,
        ],
    )
    def k(h_hbm, fb_hbm, zeros_hbm, out_hbm, fbb, rbuf, tbl, sem):
        c = lax.axis_index("c")
        s = lax.axis_index("s")
        pltpu.sync_copy(fb_hbm.at[s], fbb)

        for q in range(2):
            qq = 2 * c + q

            @pl.when(s == 0)
            def _():
                pltpu.sync_copy(zeros_hbm, tbl)

            plsc.subcore_barrier()

            def body(kk, _):
                pltpu.async_copy(
                    h_hbm.at[pl.ds(qq * NP + s * RPT + kk * 128, 128)],
                    rbuf, sem,
                ).wait()
                pltpu.sync_copy(rbuf, tbl.at[fbb.at[kk]], add=True)
                return 0

            lax.fori_loop(0, 5, body, 0)
            plsc.subcore_barrier()
            pltpu.sync_copy(
                tbl.at[pl.ds(s * rbf, rbf)],
                out_hbm.at[pl.ds(qq * FP + s * rbf, rbf)],
            )
            plsc.subcore_barrier()

    return k


def kernel(x, edge_index, edge_attr, frag_batch, frag_edge_index, dangling_edge_attr, enc_W, enc_b, enc_We, enc_be, proj_W1, proj_b1, proj_W2, proj_b2, pred_W, pred_b, pred_We, pred_be, cls_W, cls_b):
    f32 = jnp.float32

    x_pad = jnp.pad(x, ((0, NP - N), (0, DP - D)))
    src_pad = jnp.pad(edge_index[0], (0, EP - E), constant_values=N)
    dst_pad = jnp.pad(edge_index[1], (0, EP - E), constant_values=N)
    ea_pad = jnp.pad(edge_attr, ((0, EP - E), (0, 0)))
    fb_pad = jnp.pad(frag_batch, (0, NP - N), constant_values=F)

    u = jnp.concatenate([frag_edge_index[0], frag_edge_index[1]], axis=0)
    v = jnp.concatenate([frag_edge_index[1], frag_edge_index[0]], axis=0)
    u_pad = jnp.pad(u, (0, E2P - 2 * EF), constant_values=F)
    v_pad = jnp.pad(v, (0, E2P - 2 * EF), constant_values=F)
    uva_pad = jnp.pad(
        jnp.concatenate([dangling_edge_attr, dangling_edge_attr], axis=0),
        ((0, E2P - 2 * EF), (0, 0)),
    )

    Wl = jnp.pad(enc_W, ((0, 0), (0, DP - D), (0, DP - D)))
    Wl0_q = jnp.stack([Wl[0][:, i * HQ : (i + 1) * HQ] for i in range(4)])
    Wl4q = Wl.reshape(L, 4, HQ, 4, HQ).transpose(0, 1, 3, 2, 4)
    bl_q = jnp.pad(enc_b, ((0, 0), (0, DP - D))).reshape(L, 4, 1, HQ)
    Wel = jnp.pad(enc_We, ((0, 0), (0, 0), (0, DP - D)))
    Wel_q = jnp.stack(
        [Wel[:, :, i * HQ : (i + 1) * HQ] for i in range(4)], axis=1
    )
    bel_q = jnp.pad(enc_be, ((0, 0), (0, DP - D))).reshape(L, 4, 1, HQ)
    W1p = jnp.pad(proj_W1, ((0, DP - D), (0, DP - D)))
    b1p = jnp.pad(proj_b1, (0, DP - D)).reshape(1, DP)
    W2p = jnp.pad(proj_W2, ((0, DP - D), (0, DP - D)))
    b2p = jnp.pad(proj_b2, (0, DP - D)).reshape(1, DP)
    WPp = jnp.pad(pred_W, ((0, DP - D), (0, DP - D)))
    WP_q = jnp.stack([WPp[:, i * HQ : (i + 1) * HQ] for i in range(4)])
    bP_q = jnp.pad(pred_b, (0, DP - D)).reshape(4, 1, HQ)
    WePp = jnp.pad(pred_We, ((0, 0), (0, DP - D)))
    WeP_q = jnp.stack([WePp[:, i * HQ : (i + 1) * HQ] for i in range(4)])
    beP_q = jnp.pad(pred_be, (0, DP - D)).reshape(4, 1, HQ)
    cWp = jnp.pad(cls_W, ((0, DP - D), (0, 0)))
    cbp = cls_b.reshape(1, 1)

    tmpl = jnp.concatenate(
        [jnp.ones((128, 1), f32), jnp.zeros((128, 15), f32)], axis=1
    )
    zNP16 = jnp.zeros((NP, 16), f32)
    zFP16 = jnp.zeros((FP, 16), f32)
    zFPQ = jnp.zeros((FP, HQ), f32)
    zFP1 = jnp.zeros((FP, 1), f32)

    dst_w32 = dst_pad.reshape(2 * NT, 40, 128)
    src_w32 = src_pad.reshape(2 * NT, 40, 128)
    fb_w32 = fb_pad.reshape(2 * NT, 5, 64)
    v2_w32 = v_pad.reshape(2 * NT, 1, 128)
    u2_w32 = u_pad.reshape(2 * NT, 1, 128)
    src_t16 = src_pad.reshape(NT, 80, 128)
    dst_t16 = dst_pad.reshape(NT, 80, 128)
    u2_t16 = u_pad.reshape(NT, 2, 128)
    v2_t16 = v_pad.reshape(NT, 2, 128)
    fb_t16 = fb_pad.reshape(NT, 5, 128)

    degt = _sc_hist(NP, 40, 128)(dst_w32, tmpl, zNP16)
    fct = _sc_hist(FP, 5, 64)(fb_w32, tmpl, zFP16)
    fvt = _sc_hist(FP, 1, 128)(v2_w32, tmpl, zFP16)
    dinv, ss, rcnt, dinv2 = _tc_prep()(degt, fct, fvt)

    nrm = _sc_norm(EP, 40, 128)(src_w32, dst_w32, dinv)
    nrm2 = _sc_norm(E2P, 1, 128)(u2_w32, v2_w32, dinv2)

    edge_np = _sc_edge(NP, EP, 80, 128)
    gt, it = _tc_layer0(NP)(x_pad, Wl0_q, bl_q[0], ss)
    out4 = None
    for l in range(L):
        em = _tc_eemb(EP)(ea_pad, Wel_q[l], bel_q[l])
        out4 = edge_np(
            gt.reshape(4 * NP, HQ), em.reshape(4 * EP, HQ),
            src_t16, dst_t16, nrm, it.reshape(4 * NP, HQ),
        )
        if l < L - 1:
            w4 = Wl4q[l + 1]
            gt, it = _tc_layer(NP)(
                out4, out4, out4, out4, w4, w4, w4, w4, bl_q[l + 1], ss
            )

    sums = _sc_pool()(out4, fb_t16, zFPQ)
    outp, f0 = _tc_mlp()(sums.reshape(4, FP, HQ), rcnt, W1p, b1p, W2p, b2p)

    g2, i2 = _tc_layer0(FP)(outp, WP_q, bP_q, zFP1)
    em2 = _tc_eemb(E2P)(uva_pad, WeP_q, beP_q)
    pred4 = _sc_edge(FP, E2P, 2, 128)(
        g2.reshape(4 * FP, HQ), em2.reshape(4 * E2P, HQ),
        u2_t16, v2_t16, nrm2, i2.reshape(4 * FP, HQ),
    )

    fin = _tc_final()(pred4.reshape(4, FP, HQ), f0, cWp, cbp)
    logits = jnp.concatenate([fin[:F, 0], fin[:F, 1]], axis=0)
    labels = jnp.concatenate(
        [jnp.ones((F,), jnp.float32), jnp.zeros((F,), jnp.float32)], axis=0
    )
    return (logits, labels)
